# baseline (speedup 1.0000x reference)
import functools

import jax
import jax.numpy as jnp
from jax import lax
from jax.experimental import pallas as pl
from jax.experimental.pallas import tpu as pltpu
from jax.experimental.pallas import tpu_sc as plsc

VOCAB = 1000000
D = 64
SEQ = 200
BATCH = 4096
N = BATCH * SEQ

NC, NS = 2, 16
NW = NC * NS
PER_W = N // NW
SEQ_PER_CHUNK = 4
CHUNK = SEQ * SEQ_PER_CHUNK
NCHUNK = PER_W // CHUNK
GATHER_W = 100
NGATHER = CHUNK // GATHER_W


def _make_kernel():
    mesh = plsc.VectorSubcoreMesh(core_axis_name="c", subcore_axis_name="s")

    @functools.partial(
        pl.kernel,
        mesh=mesh,
        out_type=jax.ShapeDtypeStruct((N, D), jnp.float32),
        compiler_params=pltpu.CompilerParams(use_tc_tiling_on_sc=False),
        scratch_types=[
            pltpu.VMEM((NGATHER, GATHER_W), jnp.int32),
            pltpu.VMEM((CHUNK, D), jnp.float32),
            pltpu.VMEM((SEQ, D), jnp.float32),
            pltpu.SemaphoreType.DMA,
        ],
    )
    def emb_kernel(tok_hbm, xf_hbm, pos_hbm, out_hbm, idx_v, rows_v, pos_v, sem):
        wid = lax.axis_index("s") * NC + lax.axis_index("c")
        base = wid * PER_W

        pltpu.sync_copy(pos_hbm, pos_v)

        def chunk_body(g, carry):
            start = base + g * CHUNK
            row0 = pl.multiple_of(start // GATHER_W, NGATHER)
            pltpu.sync_copy(xf_hbm.at[pl.ds(row0, NGATHER)], idx_v)
            for j in range(NGATHER):
                pltpu.async_copy(
                    tok_hbm.at[idx_v.at[j]],
                    rows_v.at[pl.ds(j * GATHER_W, GATHER_W)],
                    sem,
                )
            for j in range(NGATHER):
                pltpu.make_async_copy(
                    tok_hbm.at[idx_v.at[j]],
                    rows_v.at[pl.ds(j * GATHER_W, GATHER_W)],
                    sem,
                ).wait()

            def pos_body(s, c):
                for j in range(D // 16):
                    pv = pos_v[s, pl.ds(16 * j, 16)]
                    for kk in range(SEQ_PER_CHUNK):
                        plsc.addupdate(
                            rows_v.at[kk * SEQ + s, pl.ds(16 * j, 16)], pv
                        )
                return c

            lax.fori_loop(0, SEQ, pos_body, 0)

            pltpu.sync_copy(rows_v, out_hbm.at[pl.ds(start, CHUNK)])
            return carry

        lax.fori_loop(0, NCHUNK, chunk_body, 0)

    return emb_kernel


_emb_kernel = _make_kernel()


@jax.jit
def kernel(x, token_table, pos_table):
    xf = x.reshape(N // GATHER_W, GATHER_W).astype(jnp.int32)
    out = _emb_kernel(token_table, xf, pos_table)
    return out.reshape(BATCH, SEQ, D)
